# baseline (speedup 1.0000x reference)
import functools

import jax
import jax.numpy as jnp
import numpy as np
from jax import lax
from jax.experimental import pallas as pl
from jax.experimental.pallas import tpu as pltpu
from jax.experimental.pallas import tpu_sc as plsc

_BINS = 4096
_ADIM = 8
_OUT_DIM = _BINS * (_ADIM + 1)
_BK = 128
_BS = 16
_L = 16


def _matmul_body(x_ref, w_ref, b_ref, obins_ref, ores_ref):
    k = pl.program_id(0)
    xk = x_ref[:, pl.ds(k * _BK, _BK)]
    wk = w_ref[...]
    bins_part = jnp.dot(
        xk, wk[:, :_BINS], preferred_element_type=jnp.float32
    )
    res_part = jnp.dot(
        xk.astype(jnp.bfloat16),
        wk[:, _BINS:].astype(jnp.bfloat16),
        preferred_element_type=jnp.float32,
    )

    @pl.when(k == 0)
    def _():
        obins_ref[...] = bins_part + b_ref[:, :_BINS]
        ores_ref[...] = res_part + b_ref[:, _BINS:]

    @pl.when(k != 0)
    def _():
        obins_ref[...] = obins_ref[...] + bins_part
        ores_ref[...] = ores_ref[...] + res_part


def _sc_sample_gather(
    bins_hbm,
    gmb_hbm,
    resid_hbm,
    sel_hbm,
    selres_hbm,
    binsv,
    gmbv,
    idxv,
    rowsv,
    selv,
    stagev,
    sem,
):
    nc = 2
    wid = lax.axis_index("s") * nc + lax.axis_index("c")

    @pl.when(wid < _BS)
    def _():
        t = wid
        pltpu.sync_copy(bins_hbm.at[t], binsv)
        pltpu.sync_copy(gmb_hbm.at[t], gmbv)

        lanes = lax.iota(jnp.int32, _L)
        init_val = jnp.full((_L,), -jnp.inf, jnp.float32)
        init_idx = jnp.zeros((_L,), jnp.int32)

        def body(j, carry):
            bv, bi = carry
            v = binsv[pl.ds(j * _L, _L)] + gmbv[pl.ds(j * _L, _L)]
            idx = lanes + j * _L
            upd = v > bv
            return (
                jnp.where(upd, v, bv),
                jnp.where(upd, idx, bi),
            )

        best_val, best_idx = lax.fori_loop(
            0, _BINS // _L, body, (init_val, init_idx)
        )
        cmax = plsc.cummax(best_val)
        gmaxv = plsc.cummax(lax.rev(cmax, dimensions=(0,)))
        cand = jnp.where(best_val == gmaxv, best_idx, jnp.int32(2**30))
        cm = plsc.cummax(-cand)
        sel_vec = -plsc.cummax(lax.rev(cm, dimensions=(0,)))

        for i in range(128 // _L):
            selv[pl.ds(i * _L, _L)] = sel_vec
        pltpu.sync_copy(selv, sel_hbm.at[t])

        rows_per_tok = _BINS * _ADIM // 128
        idxv[...] = sel_vec // (128 // _ADIM) + t * rows_per_tok
        pltpu.async_copy(resid_hbm.at[idxv], rowsv, sem).wait()
        off = (sel_vec % (128 // _ADIM)) * _ADIM
        col = jnp.minimum(off + lanes, jnp.int32(127))
        picked = plsc.load_gather(
            rowsv, [jnp.zeros((_L,), jnp.int32), col]
        )
        for i in range(128 // _L):
            stagev[pl.ds(i * _L, _L)] = picked
        pltpu.sync_copy(stagev, selres_hbm.at[t])


def _gumbel_noise(bs, num_bins):
    return jax.random.gumbel(
        jax.random.key(42), (bs, num_bins), jnp.float32
    )


def kernel(transformer_logits, W, b):
    batch, seq, num_bins = transformer_logits.shape
    bs = batch * seq
    x2d = transformer_logits.reshape(bs, num_bins)
    b2d = b.reshape(1, _OUT_DIM)
    gumbel = _gumbel_noise(bs, num_bins)

    nsteps = num_bins // _BK
    bins_logits, resid = pl.pallas_call(
        _matmul_body,
        grid=(nsteps,),
        in_specs=[
            pl.BlockSpec((bs, num_bins), lambda k: (0, 0)),
            pl.BlockSpec((_BK, _OUT_DIM), lambda k: (k, 0)),
            pl.BlockSpec((1, _OUT_DIM), lambda k: (0, 0)),
        ],
        out_specs=(
            pl.BlockSpec((bs, _BINS), lambda k: (0, 0)),
            pl.BlockSpec((bs, _OUT_DIM - _BINS), lambda k: (0, 0)),
        ),
        out_shape=(
            jax.ShapeDtypeStruct((bs, _BINS), jnp.float32),
            jax.ShapeDtypeStruct((bs, _OUT_DIM - _BINS), jnp.float32),
        ),
        compiler_params=pltpu.CompilerParams(
            dimension_semantics=("arbitrary",)
        ),
    )(x2d, W, b2d)

    resid_rows = resid.reshape(bs * num_bins * _ADIM // 128, 128)

    mesh = plsc.VectorSubcoreMesh(
        core_axis_name="c", subcore_axis_name="s"
    )
    sel_mat, selres = pl.kernel(
        _sc_sample_gather,
        mesh=mesh,
        out_type=(
            jax.ShapeDtypeStruct((bs, 128), jnp.int32),
            jax.ShapeDtypeStruct((bs, 128), jnp.float32),
        ),
        scratch_types=[
            pltpu.VMEM((num_bins,), jnp.float32),
            pltpu.VMEM((num_bins,), jnp.float32),
            pltpu.VMEM((_L,), jnp.int32),
            pltpu.VMEM((_L, 128), jnp.float32),
            pltpu.VMEM((128,), jnp.int32),
            pltpu.VMEM((128,), jnp.float32),
            pltpu.SemaphoreType.DMA,
        ],
        compiler_params=pltpu.CompilerParams(needs_layout_passes=False),
    )(bins_logits, gumbel, resid_rows)

    sel = sel_mat[:, :1]
    selres = selres[:, :_ADIM]

    return (
        sel.reshape(batch, seq, 1),
        selres.reshape(batch, seq, _ADIM),
        resid.reshape(batch, seq, num_bins, _ADIM),
        bins_logits.reshape(batch, seq, num_bins),
    )


# speedup vs baseline: 1.0805x; 1.0805x over previous
import functools

import jax
import jax.numpy as jnp
from jax.experimental import pallas as pl
from jax.experimental.pallas import tpu as pltpu

_BINS = 4096
_ADIM = 8
_OUT_DIM = _BINS * (_ADIM + 1)
_BK = 128
_BS = 16

def _gumbel_noise():
    return jax.random.gumbel(
        jax.random.key(42), (_BS, _BINS), jnp.float32
    )


def _fused_body(
    x_ref,
    w_ref,
    b_ref,
    gmb_ref,
    obins_ref,
    ores_ref,
    osel_ref,
    oselres_ref,
    *,
    nsteps,
):
    k = pl.program_id(0)
    xk = x_ref[:, pl.ds(k * _BK, _BK)]
    wk = w_ref[...]
    bins_part = jnp.dot(
        xk, wk[:, :_BINS], preferred_element_type=jnp.float32
    )
    res_part = jnp.dot(
        xk.astype(jnp.bfloat16),
        wk[:, _BINS:].astype(jnp.bfloat16),
        preferred_element_type=jnp.float32,
    )

    @pl.when(k == 0)
    def _():
        obins_ref[...] = bins_part + b_ref[:, :_BINS]
        ores_ref[...] = res_part + b_ref[:, _BINS:]

    @pl.when(k != 0)
    def _():
        obins_ref[...] = obins_ref[...] + bins_part
        ores_ref[...] = ores_ref[...] + res_part

    @pl.when(k == nsteps - 1)
    def _():
        z = obins_ref[...] + gmb_ref[...]
        sel = jnp.argmax(z, axis=-1).astype(jnp.int32)
        osel_ref[...] = sel[:, None]
        cols = jax.lax.broadcasted_iota(jnp.int32, (_BS, _BINS * _ADIM), 1)
        resid = ores_ref[...]
        parts = []
        for c in range(_ADIM):
            m = cols == sel[:, None] * _ADIM + c
            parts.append(
                jnp.sum(jnp.where(m, resid, 0.0), axis=1, keepdims=True)
            )
        oselres_ref[...] = jnp.concatenate(parts, axis=1)


def kernel(transformer_logits, W, b):
    batch, seq, num_bins = transformer_logits.shape
    bs = batch * seq
    x2d = transformer_logits.reshape(bs, num_bins)
    b2d = b.reshape(1, _OUT_DIM)
    gumbel = _gumbel_noise()

    nsteps = num_bins // _BK
    bins_logits, resid, sel, selres = pl.pallas_call(
        functools.partial(_fused_body, nsteps=nsteps),
        grid=(nsteps,),
        in_specs=[
            pl.BlockSpec((bs, num_bins), lambda k: (0, 0)),
            pl.BlockSpec((_BK, _OUT_DIM), lambda k: (k, 0)),
            pl.BlockSpec((1, _OUT_DIM), lambda k: (0, 0)),
            pl.BlockSpec((bs, _BINS), lambda k: (0, 0)),
        ],
        out_specs=(
            pl.BlockSpec((bs, _BINS), lambda k: (0, 0)),
            pl.BlockSpec((bs, _OUT_DIM - _BINS), lambda k: (0, 0)),
            pl.BlockSpec((bs, 1), lambda k: (0, 0)),
            pl.BlockSpec((bs, _ADIM), lambda k: (0, 0)),
        ),
        out_shape=(
            jax.ShapeDtypeStruct((bs, _BINS), jnp.float32),
            jax.ShapeDtypeStruct((bs, _OUT_DIM - _BINS), jnp.float32),
            jax.ShapeDtypeStruct((bs, 1), jnp.int32),
            jax.ShapeDtypeStruct((bs, _ADIM), jnp.float32),
        ),
        compiler_params=pltpu.CompilerParams(
            dimension_semantics=("arbitrary",)
        ),
    )(x2d, W, b2d, gumbel)

    return (
        sel.reshape(batch, seq, 1),
        selres.reshape(batch, seq, _ADIM),
        resid.reshape(batch, seq, num_bins, _ADIM),
        bins_logits.reshape(batch, seq, num_bins),
    )


# speedup vs baseline: 1.0889x; 1.0078x over previous
import functools

import jax
import jax.numpy as jnp
from jax.experimental import pallas as pl
from jax.experimental.pallas import tpu as pltpu

_BINS = 4096
_ADIM = 8
_OUT_DIM = _BINS * (_ADIM + 1)
_BK = 128
_BS = 16

def _gumbel_noise():
    return jax.random.gumbel(
        jax.random.key(42), (_BS, _BINS), jnp.float32
    )


_HALFW = _OUT_DIM // 2
_RES0 = _HALFW - _BINS


def _fused_body(
    x_ref,
    w_ref,
    b_ref,
    gmb_ref,
    obins_ref,
    ores_ref,
    osel_ref,
    oselres_ref,
    *,
    nsteps,
):
    k = pl.program_id(0)
    j = pl.program_id(1)
    xk = x_ref[:, pl.ds(k * _BK, _BK)]
    xk16 = xk.astype(jnp.bfloat16)
    wk = w_ref[...]

    @pl.when(j == 0)
    def _():
        bins_part = jnp.dot(
            xk, wk[:, :_BINS], preferred_element_type=jnp.float32
        )
        res_part = jnp.dot(
            xk16,
            wk[:, _BINS:].astype(jnp.bfloat16),
            preferred_element_type=jnp.float32,
        )

        @pl.when(k == 0)
        def _():
            obins_ref[...] = bins_part + b_ref[:, :_BINS]
            ores_ref[:, :_RES0] = res_part + b_ref[:, _BINS:_HALFW]

        @pl.when(k != 0)
        def _():
            obins_ref[...] = obins_ref[...] + bins_part
            ores_ref[:, :_RES0] = ores_ref[:, :_RES0] + res_part

    @pl.when(j == 1)
    def _():
        res_part = jnp.dot(
            xk16,
            wk.astype(jnp.bfloat16),
            preferred_element_type=jnp.float32,
        )

        @pl.when(k == 0)
        def _():
            ores_ref[:, _RES0:] = res_part + b_ref[:, _HALFW:]

        @pl.when(k != 0)
        def _():
            ores_ref[:, _RES0:] = ores_ref[:, _RES0:] + res_part

    @pl.when((k == nsteps - 1) & (j == 1))
    def _():
        z = obins_ref[...] + gmb_ref[...]
        sel = jnp.argmax(z, axis=-1).astype(jnp.int32)
        osel_ref[...] = sel[:, None]
        cols = jax.lax.broadcasted_iota(jnp.int32, (_BS, _BINS * _ADIM), 1)
        resid = ores_ref[...]
        parts = []
        for c in range(_ADIM):
            m = cols == sel[:, None] * _ADIM + c
            parts.append(
                jnp.sum(jnp.where(m, resid, 0.0), axis=1, keepdims=True)
            )
        oselres_ref[...] = jnp.concatenate(parts, axis=1)


def kernel(transformer_logits, W, b):
    batch, seq, num_bins = transformer_logits.shape
    bs = batch * seq
    x2d = transformer_logits.reshape(bs, num_bins)
    b2d = b.reshape(1, _OUT_DIM)
    gumbel = _gumbel_noise()

    nsteps = num_bins // _BK
    bins_logits, resid, sel, selres = pl.pallas_call(
        functools.partial(_fused_body, nsteps=nsteps),
        grid=(nsteps, 2),
        in_specs=[
            pl.BlockSpec((bs, num_bins), lambda k, j: (0, 0)),
            pl.BlockSpec((_BK, _HALFW), lambda k, j: (k, j)),
            pl.BlockSpec((1, _OUT_DIM), lambda k, j: (0, 0)),
            pl.BlockSpec((bs, _BINS), lambda k, j: (0, 0)),
        ],
        out_specs=(
            pl.BlockSpec((bs, _BINS), lambda k, j: (0, 0)),
            pl.BlockSpec((bs, _OUT_DIM - _BINS), lambda k, j: (0, 0)),
            pl.BlockSpec((bs, 1), lambda k, j: (0, 0)),
            pl.BlockSpec((bs, _ADIM), lambda k, j: (0, 0)),
        ),
        out_shape=(
            jax.ShapeDtypeStruct((bs, _BINS), jnp.float32),
            jax.ShapeDtypeStruct((bs, _OUT_DIM - _BINS), jnp.float32),
            jax.ShapeDtypeStruct((bs, 1), jnp.int32),
            jax.ShapeDtypeStruct((bs, _ADIM), jnp.float32),
        ),
        compiler_params=pltpu.CompilerParams(
            dimension_semantics=("arbitrary", "arbitrary")
        ),
    )(x2d, W, b2d, gumbel)

    return (
        sel.reshape(batch, seq, 1),
        selres.reshape(batch, seq, _ADIM),
        resid.reshape(batch, seq, num_bins, _ADIM),
        bins_logits.reshape(batch, seq, num_bins),
    )
